# Initial kernel scaffold; baseline (speedup 1.0000x reference)
import jax
import jax.numpy as jnp
from jax.experimental import pallas as pl


def kernel(v_size, c_size, v_edge_index, c_edge_index, p_edge_index, n_edge_index, v_emb, c_emb, params):
    raise NotImplementedError("write your pallas kernel here")


# speedup vs baseline: 1.8237x; 1.8237x over previous
import functools

import jax
import jax.numpy as jnp
from jax import lax
from jax.experimental import pallas as pl
from jax.experimental.pallas import tpu as pltpu
from jax.experimental.pallas import tpu_sc as plsc

DIM = 256
HALF = 128
N_LAYERS = 3
N_ITER = 3
NV = 10000
NC_NODES = 10000
E = 160000
EP = 80000
EN = 80000

SC_CORES = 2
SC_SUBCORES = 16
LANES = 16
CHUNK = 128
NCHUNK = 40
EPAD = SC_SUBCORES * NCHUNK * CHUNK
ACC_ROWS = 10112

RB = 1000
NB = NV // RB


def _feat_body(x_ref, w_ref, b_ref, out_ref):
    x = x_ref[...]
    for i in range(N_LAYERS):
        x = jnp.dot(x, w_ref[0, i], preferred_element_type=jnp.float32)
        x = x + b_ref[0, i][None, :]
        if i < N_LAYERS - 1:
            x = jnp.maximum(x, 0.0)
    out_ref[0, 0] = x[:, :HALF]
    out_ref[0, 1] = x[:, HALF:]


def _feat_mlps(emb, ws, bs):
    n = emb.shape[0]
    return pl.pallas_call(
        _feat_body,
        grid=(2, n // RB),
        in_specs=[
            pl.BlockSpec((RB, DIM), lambda m, r: (r, 0)),
            pl.BlockSpec((1, N_LAYERS, DIM, DIM), lambda m, r: (m, 0, 0, 0)),
            pl.BlockSpec((1, N_LAYERS, DIM), lambda m, r: (m, 0, 0)),
        ],
        out_specs=pl.BlockSpec((1, 2, RB, HALF), lambda m, r: (m, 0, r, 0)),
        out_shape=jax.ShapeDtypeStruct((2, 2, n, HALF), jnp.float32),
    )(emb, ws, bs)


def _upd_body(x_ref, p0_ref, p1_ref, n0_ref, n1_ref, w1_ref, b1_ref,
              w23_ref, b23_ref, out_ref):
    h = jnp.dot(x_ref[...], w1_ref[0:DIM], preferred_element_type=jnp.float32)
    h += jnp.dot(p0_ref[0], w1_ref[DIM:DIM + HALF],
                 preferred_element_type=jnp.float32)
    h += jnp.dot(p1_ref[0], w1_ref[DIM + HALF:2 * DIM],
                 preferred_element_type=jnp.float32)
    h += jnp.dot(n0_ref[0], w1_ref[2 * DIM:2 * DIM + HALF],
                 preferred_element_type=jnp.float32)
    h += jnp.dot(n1_ref[0], w1_ref[2 * DIM + HALF:3 * DIM],
                 preferred_element_type=jnp.float32)
    h += b1_ref[0][None, :]
    h = jnp.maximum(h, 0.0)
    h = jnp.dot(h, w23_ref[0], preferred_element_type=jnp.float32)
    h = jnp.maximum(h + b23_ref[0][None, :], 0.0)
    h = jnp.dot(h, w23_ref[1], preferred_element_type=jnp.float32)
    out_ref[...] = h + b23_ref[1][None, :]


def _upd_mlp(emb, p_aggr, n_aggr, w1, b1, w23, b23):
    n = emb.shape[0]
    half_spec0 = pl.BlockSpec((1, RB, HALF), lambda r: (0, r, 0))
    half_spec1 = pl.BlockSpec((1, RB, HALF), lambda r: (1, r, 0))
    return pl.pallas_call(
        _upd_body,
        grid=(n // RB,),
        in_specs=[
            pl.BlockSpec((RB, DIM), lambda r: (r, 0)),
            half_spec0, half_spec1, half_spec0, half_spec1,
            pl.BlockSpec((3 * DIM, DIM), lambda r: (0, 0)),
            pl.BlockSpec((1, DIM), lambda r: (0, 0)),
            pl.BlockSpec((2, DIM, DIM), lambda r: (0, 0, 0)),
            pl.BlockSpec((2, DIM), lambda r: (0, 0)),
        ],
        out_specs=pl.BlockSpec((RB, DIM), lambda r: (r, 0)),
        out_shape=jax.ShapeDtypeStruct((n, DIM), jnp.float32),
    )(emb, p_aggr, p_aggr, n_aggr, n_aggr, w1, b1, w23, b23)


_MESH = plsc.VectorSubcoreMesh(core_axis_name="c", subcore_axis_name="s")


@functools.partial(
    pl.kernel,
    mesh=_MESH,
    out_type=[
        jax.ShapeDtypeStruct((SC_SUBCORES, NCHUNK, CHUNK), jnp.int32)
        for _ in range(4)
    ],
    scratch_types=[
        pltpu.VMEM((NCHUNK, CHUNK), jnp.int32),
        pltpu.VMEM((NCHUNK, CHUNK), jnp.int32),
        pltpu.VMEM((NCHUNK, CHUNK), jnp.int32),
        pltpu.SemaphoreType.DMA,
    ],
)
def _compose_kernel(v_edge_hbm, c_edge_hbm, p_sel_hbm, n_sel_hbm,
                    vsrc_p_hbm, cdst_p_hbm, vsrc_n_hbm, cdst_n_hbm,
                    sel_v, va_v, ca_v, sem):
    cid = lax.axis_index("c")
    sid = lax.axis_index("s")

    def run(sel_hbm, vsrc_hbm, cdst_hbm):
        pltpu.sync_copy(sel_hbm.at[sid], sel_v)

        def chunk(j, carry):
            pltpu.async_copy(v_edge_hbm.at[sel_v.at[j]], va_v.at[j], sem).wait()
            pltpu.async_copy(c_edge_hbm.at[sel_v.at[j]], ca_v.at[j], sem).wait()
            return carry

        lax.fori_loop(0, NCHUNK, chunk, 0)
        pltpu.sync_copy(va_v, vsrc_hbm.at[sid])
        pltpu.sync_copy(ca_v, cdst_hbm.at[sid])

    @pl.when(cid == 0)
    def _():
        run(p_sel_hbm, vsrc_p_hbm, cdst_p_hbm)

    @pl.when(cid == 1)
    def _():
        run(n_sel_hbm, vsrc_n_hbm, cdst_n_hbm)


@functools.partial(
    pl.kernel,
    mesh=_MESH,
    out_type=jax.ShapeDtypeStruct((SC_CORES, NC_NODES, HALF), jnp.float32),
    scratch_types=[
        pltpu.VMEM((NCHUNK, CHUNK), jnp.int32),
        pltpu.VMEM((NCHUNK, CHUNK), jnp.int32),
        pltpu.VMEM((CHUNK, HALF), jnp.float32),
        pltpu.VMEM((CHUNK, HALF), jnp.float32),
        pltpu.VMEM_SHARED((ACC_ROWS, HALF), jnp.float32),
        pltpu.SemaphoreType.DMA,
    ],
)
def _aggr_kernel(feat_hbm, src_hbm, dst_hbm, zeros_hbm, out_hbm,
                 src_v, dst_v, buf_v, zbuf_v, acc_sh, sem):
    cid = lax.axis_index("c")
    sid = lax.axis_index("s")

    pltpu.sync_copy(src_hbm.at[sid], src_v)
    pltpu.sync_copy(dst_hbm.at[sid], dst_v)
    pltpu.sync_copy(zeros_hbm, zbuf_v)

    src_off = cid * NC_NODES
    sub_base = sid * (NCHUNK * CHUNK)
    lane = lax.iota(jnp.int32, LANES)

    def fix(i, carry):
        j = i // (CHUNK // LANES)
        k = i % (CHUNK // LANES)
        src_v[j, pl.ds(k * LANES, LANES)] = (
            src_v[j, pl.ds(k * LANES, LANES)] + src_off)
        pos = sub_base + j * CHUNK + k * LANES + lane
        d = dst_v[j, pl.ds(k * LANES, LANES)]
        dst_v[j, pl.ds(k * LANES, LANES)] = jnp.where(
            pos < EP, d, NC_NODES)
        return carry

    lax.fori_loop(0, NCHUNK * (CHUNK // LANES), fix, 0)

    zrows = ACC_ROWS // SC_SUBCORES
    zbase = sid * zrows
    nfull = zrows // CHUNK
    for t in range(nfull):
        pltpu.sync_copy(zbuf_v, acc_sh.at[pl.ds(zbase + t * CHUNK, CHUNK)])
    rem = zrows - nfull * CHUNK
    if rem:
        pltpu.sync_copy(zbuf_v.at[pl.ds(0, rem)],
                        acc_sh.at[pl.ds(zbase + nfull * CHUNK, rem)])

    plsc.subcore_barrier()

    def chunk(j, carry):
        pltpu.async_copy(feat_hbm.at[src_v.at[j]], buf_v, sem).wait()
        pltpu.sync_copy(buf_v, acc_sh.at[dst_v.at[j]], add=True)
        return carry

    lax.fori_loop(0, NCHUNK, chunk, 0)

    plsc.subcore_barrier()

    nfull_out = NC_NODES // CHUNK
    for t in range((nfull_out + SC_SUBCORES - 1) // SC_SUBCORES):
        g = sid + t * SC_SUBCORES

        @pl.when(g < nfull_out)
        def _():
            base = g * CHUNK
            pltpu.sync_copy(acc_sh.at[pl.ds(base, CHUNK)], buf_v)
            pltpu.sync_copy(buf_v, out_hbm.at[cid, pl.ds(base, CHUNK)])

    tail = NC_NODES - nfull_out * CHUNK
    @pl.when(sid == nfull_out % SC_SUBCORES)
    def _():
        base = nfull_out * CHUNK
        pltpu.sync_copy(acc_sh.at[pl.ds(base, tail)],
                        buf_v.at[pl.ds(0, tail)])
        pltpu.sync_copy(buf_v.at[pl.ds(0, tail)],
                        out_hbm.at[cid, pl.ds(base, tail)])


def _stack_mlp(params):
    ws = jnp.stack([w for w, _ in params])
    bs = jnp.stack([b for _, b in params])
    return ws, bs


def kernel(v_size, c_size, v_edge_index, c_edge_index, p_edge_index,
           n_edge_index, v_emb, c_emb, params):
    f32 = jnp.float32
    v_emb = v_emb * (jnp.asarray(v_size) == NV).astype(f32)
    c_emb = c_emb * (jnp.asarray(c_size) == NC_NODES).astype(f32)

    def pad_sel(sel):
        return jnp.concatenate(
            [sel, jnp.zeros((EPAD - sel.shape[0],), jnp.int32)]
        ).reshape(SC_SUBCORES, NCHUNK, CHUNK)

    p_sel = pad_sel(p_edge_index)
    n_sel = pad_sel(n_edge_index)
    vsrc_p, cdst_p, vsrc_n, cdst_n = _compose_kernel(
        v_edge_index, c_edge_index, p_sel, n_sel)

    zeros_blk = jnp.zeros((CHUNK, HALF), f32)

    wv_p, bv_p = _stack_mlp(params["p_v2c"])
    wv_n, bv_n = _stack_mlp(params["n_v2c"])
    wc_p, bc_p = _stack_mlp(params["p_c2v"])
    wc_n, bc_n = _stack_mlp(params["n_c2v"])
    wv = jnp.stack([wv_p, wv_n])
    bv = jnp.stack([bv_p, bv_n])
    wc = jnp.stack([wc_p, wc_n])
    bc = jnp.stack([bc_p, bc_n])

    def upd_weights(p):
        (w1, b1), (w2, b2), (w3, b3) = p
        return (w1, b1[None, :], jnp.stack([w2, w3]), jnp.stack([b2, b3]))

    cw = upd_weights(params["c_upd"])
    vw = upd_weights(params["v_upd"])

    v_embs = [v_emb]
    c_embs = [c_emb]
    for _ in range(N_ITER):
        feat_v = _feat_mlps(v_emb, wv, bv)
        feat_c = _feat_mlps(c_emb, wc, bc)

        p_v2c = feat_v[0].reshape(SC_CORES * NV, HALF)
        n_v2c = feat_v[1].reshape(SC_CORES * NV, HALF)
        p_c2v = feat_c[0].reshape(SC_CORES * NC_NODES, HALF)
        n_c2v = feat_c[1].reshape(SC_CORES * NC_NODES, HALF)

        p_v2c_aggr = _aggr_kernel(p_v2c, vsrc_p, cdst_p, zeros_blk)
        n_v2c_aggr = _aggr_kernel(n_v2c, vsrc_n, cdst_n, zeros_blk)
        p_c2v_aggr = _aggr_kernel(p_c2v, cdst_p, vsrc_p, zeros_blk)
        n_c2v_aggr = _aggr_kernel(n_c2v, cdst_n, vsrc_n, zeros_blk)

        c_emb = _upd_mlp(c_emb, p_v2c_aggr, n_v2c_aggr, *cw)
        c_embs.append(c_emb)
        v_emb = _upd_mlp(v_emb, p_c2v_aggr, n_c2v_aggr, *vw)
        v_embs.append(v_emb)

    return (jnp.stack(v_embs), jnp.stack(c_embs))


# speedup vs baseline: 2.1241x; 1.1647x over previous
import functools

import jax
import jax.numpy as jnp
from jax import lax
from jax.experimental import pallas as pl
from jax.experimental.pallas import tpu as pltpu
from jax.experimental.pallas import tpu_sc as plsc

DIM = 256
HALF = 128
N_LAYERS = 3
N_ITER = 3
NV = 10000
NC_NODES = 10000
E = 160000
EP = 80000
EN = 80000

SC_CORES = 2
SC_SUBCORES = 16
LANES = 16
CHUNK = 128
NCHUNK = 40
EPAD = SC_SUBCORES * NCHUNK * CHUNK
ACC_ROWS = 10112

RB = 1000
NB = NV // RB


def _feat_body(x_ref, w_ref, b_ref, out_ref):
    x = x_ref[...]
    for i in range(N_LAYERS):
        x = jnp.dot(x, w_ref[0, i], preferred_element_type=jnp.float32)
        x = x + b_ref[0, i][None, :]
        if i < N_LAYERS - 1:
            x = jnp.maximum(x, 0.0)
    out_ref[0, 0] = x[:, :HALF]
    out_ref[0, 1] = x[:, HALF:]


def _feat_mlps(emb, ws, bs):
    n = emb.shape[0]
    return pl.pallas_call(
        _feat_body,
        grid=(2, n // RB),
        in_specs=[
            pl.BlockSpec((RB, DIM), lambda m, r: (r, 0)),
            pl.BlockSpec((1, N_LAYERS, DIM, DIM), lambda m, r: (m, 0, 0, 0)),
            pl.BlockSpec((1, N_LAYERS, DIM), lambda m, r: (m, 0, 0)),
        ],
        out_specs=pl.BlockSpec((1, 2, RB, HALF), lambda m, r: (m, 0, r, 0)),
        out_shape=jax.ShapeDtypeStruct((2, 2, n, HALF), jnp.float32),
    )(emb, ws, bs)


def _upd_body(x_ref, p0_ref, p1_ref, n0_ref, n1_ref, w1_ref, b1_ref,
              w23_ref, b23_ref, out_ref):
    h = jnp.dot(x_ref[...], w1_ref[0:DIM], preferred_element_type=jnp.float32)
    h += jnp.dot(p0_ref[0], w1_ref[DIM:DIM + HALF],
                 preferred_element_type=jnp.float32)
    h += jnp.dot(p1_ref[0], w1_ref[DIM + HALF:2 * DIM],
                 preferred_element_type=jnp.float32)
    h += jnp.dot(n0_ref[0], w1_ref[2 * DIM:2 * DIM + HALF],
                 preferred_element_type=jnp.float32)
    h += jnp.dot(n1_ref[0], w1_ref[2 * DIM + HALF:3 * DIM],
                 preferred_element_type=jnp.float32)
    h += b1_ref[0][None, :]
    h = jnp.maximum(h, 0.0)
    h = jnp.dot(h, w23_ref[0], preferred_element_type=jnp.float32)
    h = jnp.maximum(h + b23_ref[0][None, :], 0.0)
    h = jnp.dot(h, w23_ref[1], preferred_element_type=jnp.float32)
    out_ref[...] = h + b23_ref[1][None, :]


def _upd_mlp(emb, p_aggr, n_aggr, w1, b1, w23, b23):
    n = emb.shape[0]
    half_spec0 = pl.BlockSpec((1, RB, HALF), lambda r: (0, r, 0))
    half_spec1 = pl.BlockSpec((1, RB, HALF), lambda r: (1, r, 0))
    return pl.pallas_call(
        _upd_body,
        grid=(n // RB,),
        in_specs=[
            pl.BlockSpec((RB, DIM), lambda r: (r, 0)),
            half_spec0, half_spec1, half_spec0, half_spec1,
            pl.BlockSpec((3 * DIM, DIM), lambda r: (0, 0)),
            pl.BlockSpec((1, DIM), lambda r: (0, 0)),
            pl.BlockSpec((2, DIM, DIM), lambda r: (0, 0, 0)),
            pl.BlockSpec((2, DIM), lambda r: (0, 0)),
        ],
        out_specs=pl.BlockSpec((RB, DIM), lambda r: (r, 0)),
        out_shape=jax.ShapeDtypeStruct((n, DIM), jnp.float32),
    )(emb, p_aggr, p_aggr, n_aggr, n_aggr, w1, b1, w23, b23)


_MESH = plsc.VectorSubcoreMesh(core_axis_name="c", subcore_axis_name="s")


@functools.partial(
    pl.kernel,
    mesh=_MESH,
    out_type=[
        jax.ShapeDtypeStruct((SC_SUBCORES, NCHUNK, CHUNK), jnp.int32)
        for _ in range(4)
    ],
    scratch_types=[
        pltpu.VMEM((NCHUNK, CHUNK), jnp.int32),
        pltpu.VMEM((NCHUNK, CHUNK), jnp.int32),
        pltpu.VMEM((NCHUNK, CHUNK), jnp.int32),
        pltpu.SemaphoreType.DMA,
    ],
)
def _compose_kernel(v_edge_hbm, c_edge_hbm, p_sel_hbm, n_sel_hbm,
                    vsrc_p_hbm, cdst_p_hbm, vsrc_n_hbm, cdst_n_hbm,
                    sel_v, va_v, ca_v, sem):
    cid = lax.axis_index("c")
    sid = lax.axis_index("s")

    def run(sel_hbm, vsrc_hbm, cdst_hbm):
        pltpu.sync_copy(sel_hbm.at[sid], sel_v)

        def chunk(j, carry):
            pltpu.async_copy(v_edge_hbm.at[sel_v.at[j]], va_v.at[j], sem).wait()
            pltpu.async_copy(c_edge_hbm.at[sel_v.at[j]], ca_v.at[j], sem).wait()
            return carry

        lax.fori_loop(0, NCHUNK, chunk, 0)
        pltpu.sync_copy(va_v, vsrc_hbm.at[sid])
        pltpu.sync_copy(ca_v, cdst_hbm.at[sid])

    @pl.when(cid == 0)
    def _():
        run(p_sel_hbm, vsrc_p_hbm, cdst_p_hbm)

    @pl.when(cid == 1)
    def _():
        run(n_sel_hbm, vsrc_n_hbm, cdst_n_hbm)


@functools.partial(
    pl.kernel,
    mesh=_MESH,
    out_type=jax.ShapeDtypeStruct((SC_CORES, NC_NODES, HALF), jnp.float32),
    scratch_types=[
        pltpu.VMEM((NCHUNK, CHUNK), jnp.int32),
        pltpu.VMEM((NCHUNK, CHUNK), jnp.int32),
        pltpu.VMEM((2, CHUNK, HALF), jnp.float32),
        pltpu.VMEM_SHARED((ACC_ROWS, HALF), jnp.float32),
        pltpu.SemaphoreType.DMA,
        pltpu.SemaphoreType.DMA,
        pltpu.SemaphoreType.DMA,
        pltpu.SemaphoreType.DMA,
        pltpu.SemaphoreType.DMA,
        pltpu.SemaphoreType.DMA,
    ],
)
def _aggr_kernel(feat_hbm, src_hbm, dst_hbm, zeros_hbm, out_hbm,
                 src_v, dst_v, bufs_v, acc_sh,
                 isem, zsem, gsem0, gsem1, ssem0, ssem1):
    cid = lax.axis_index("c")
    sid = lax.axis_index("s")
    gsem = (gsem0, gsem1)
    ssem = (ssem0, ssem1)

    d_src = pltpu.async_copy(src_hbm.at[sid], src_v, isem)
    d_dst = pltpu.async_copy(dst_hbm.at[sid], dst_v, isem)
    zbuf = bufs_v.at[1]
    pltpu.async_copy(zeros_hbm, zbuf, zsem).wait()
    zrows = ACC_ROWS // SC_SUBCORES
    zbase = sid * zrows
    zd = [
        pltpu.async_copy(zbuf, acc_sh.at[pl.ds(zbase + t * CHUNK, CHUNK)],
                         zsem)
        for t in range(zrows // CHUNK)
    ]
    zrem = zrows % CHUNK
    zd.append(pltpu.async_copy(
        zbuf.at[pl.ds(0, zrem)],
        acc_sh.at[pl.ds(zbase + (zrows // CHUNK) * CHUNK, zrem)], zsem))
    d_src.wait()
    d_dst.wait()

    src_off = cid * NC_NODES
    sub_base = sid * (NCHUNK * CHUNK)
    lane = lax.iota(jnp.int32, LANES)

    def fix(i, carry):
        j = i // (CHUNK // LANES)
        k = i % (CHUNK // LANES)
        src_v[j, pl.ds(k * LANES, LANES)] = (
            src_v[j, pl.ds(k * LANES, LANES)] + src_off)
        pos = sub_base + j * CHUNK + k * LANES + lane
        d = dst_v[j, pl.ds(k * LANES, LANES)]
        dst_v[j, pl.ds(k * LANES, LANES)] = jnp.where(
            pos < EP, d, NC_NODES)
        return carry

    lax.fori_loop(0, NCHUNK * (CHUNK // LANES), fix, 0)

    dg = {}
    ds = {}
    dg[0] = pltpu.async_copy(feat_hbm.at[src_v.at[0]], bufs_v.at[0], gsem[0])
    for d in zd:
        d.wait()
    plsc.subcore_barrier()

    for j in range(NCHUNK):
        b = j % 2
        if j + 1 < NCHUNK:
            if j - 1 >= 0:
                ds[j - 1].wait()
            dg[j + 1] = pltpu.async_copy(
                feat_hbm.at[src_v.at[j + 1]], bufs_v.at[1 - b],
                gsem[1 - b])
        dg[j].wait()
        ds[j] = pltpu.async_copy(
            bufs_v.at[b], acc_sh.at[dst_v.at[j]], ssem[b], add=True)
    ds[NCHUNK - 2].wait()
    ds[NCHUNK - 1].wait()

    plsc.subcore_barrier()

    nfull_out = NC_NODES // CHUNK
    for t in range((nfull_out + SC_SUBCORES - 1) // SC_SUBCORES):
        g = sid + t * SC_SUBCORES
        stage = bufs_v.at[t % 2]

        @pl.when(g < nfull_out)
        def _():
            base = g * CHUNK
            pltpu.sync_copy(acc_sh.at[pl.ds(base, CHUNK)], stage)
            pltpu.sync_copy(stage, out_hbm.at[cid, pl.ds(base, CHUNK)])

    tail = NC_NODES - nfull_out * CHUNK
    @pl.when(sid == nfull_out % SC_SUBCORES)
    def _():
        base = nfull_out * CHUNK
        pltpu.sync_copy(acc_sh.at[pl.ds(base, tail)],
                        bufs_v.at[0, pl.ds(0, tail)])
        pltpu.sync_copy(bufs_v.at[0, pl.ds(0, tail)],
                        out_hbm.at[cid, pl.ds(base, tail)])


def _stack_mlp(params):
    ws = jnp.stack([w for w, _ in params])
    bs = jnp.stack([b for _, b in params])
    return ws, bs


def kernel(v_size, c_size, v_edge_index, c_edge_index, p_edge_index,
           n_edge_index, v_emb, c_emb, params):
    f32 = jnp.float32
    v_emb = v_emb * (jnp.asarray(v_size) == NV).astype(f32)
    c_emb = c_emb * (jnp.asarray(c_size) == NC_NODES).astype(f32)

    def pad_sel(sel):
        return jnp.concatenate(
            [sel, jnp.zeros((EPAD - sel.shape[0],), jnp.int32)]
        ).reshape(SC_SUBCORES, NCHUNK, CHUNK)

    p_sel = pad_sel(p_edge_index)
    n_sel = pad_sel(n_edge_index)
    vsrc_p, cdst_p, vsrc_n, cdst_n = _compose_kernel(
        v_edge_index, c_edge_index, p_sel, n_sel)

    zeros_blk = jnp.zeros((CHUNK, HALF), f32)

    wv_p, bv_p = _stack_mlp(params["p_v2c"])
    wv_n, bv_n = _stack_mlp(params["n_v2c"])
    wc_p, bc_p = _stack_mlp(params["p_c2v"])
    wc_n, bc_n = _stack_mlp(params["n_c2v"])
    wv = jnp.stack([wv_p, wv_n])
    bv = jnp.stack([bv_p, bv_n])
    wc = jnp.stack([wc_p, wc_n])
    bc = jnp.stack([bc_p, bc_n])

    def upd_weights(p):
        (w1, b1), (w2, b2), (w3, b3) = p
        return (w1, b1[None, :], jnp.stack([w2, w3]), jnp.stack([b2, b3]))

    cw = upd_weights(params["c_upd"])
    vw = upd_weights(params["v_upd"])

    v_embs = [v_emb]
    c_embs = [c_emb]
    for _ in range(N_ITER):
        feat_v = _feat_mlps(v_emb, wv, bv)
        feat_c = _feat_mlps(c_emb, wc, bc)

        p_v2c = feat_v[0].reshape(SC_CORES * NV, HALF)
        n_v2c = feat_v[1].reshape(SC_CORES * NV, HALF)
        p_c2v = feat_c[0].reshape(SC_CORES * NC_NODES, HALF)
        n_c2v = feat_c[1].reshape(SC_CORES * NC_NODES, HALF)

        p_v2c_aggr = _aggr_kernel(p_v2c, vsrc_p, cdst_p, zeros_blk)
        n_v2c_aggr = _aggr_kernel(n_v2c, vsrc_n, cdst_n, zeros_blk)
        p_c2v_aggr = _aggr_kernel(p_c2v, cdst_p, vsrc_p, zeros_blk)
        n_c2v_aggr = _aggr_kernel(n_c2v, cdst_n, vsrc_n, zeros_blk)

        c_emb = _upd_mlp(c_emb, p_v2c_aggr, n_v2c_aggr, *cw)
        c_embs.append(c_emb)
        v_emb = _upd_mlp(v_emb, p_c2v_aggr, n_c2v_aggr, *vw)
        v_embs.append(v_emb)

    return (jnp.stack(v_embs), jnp.stack(c_embs))


# speedup vs baseline: 2.1324x; 1.0039x over previous
import functools

import jax
import jax.numpy as jnp
from jax import lax
from jax.experimental import pallas as pl
from jax.experimental.pallas import tpu as pltpu
from jax.experimental.pallas import tpu_sc as plsc

DIM = 256
HALF = 128
N_LAYERS = 3
N_ITER = 3
NV = 10000
NC_NODES = 10000
E = 160000
EP = 80000
EN = 80000

SC_CORES = 2
SC_SUBCORES = 16
LANES = 16
CHUNK = 128
NCHUNK = 40
EPAD = SC_SUBCORES * NCHUNK * CHUNK
ACC_ROWS = 10112

RB = 1000
NB = NV // RB


def _feat_body(x_ref, w_ref, b_ref, out_ref):
    x = x_ref[...]
    for i in range(N_LAYERS):
        x = jnp.dot(x, w_ref[0, i], preferred_element_type=jnp.float32)
        x = x + b_ref[0, i][None, :]
        if i < N_LAYERS - 1:
            x = jnp.maximum(x, 0.0)
    out_ref[0, 0] = x[:, :HALF]
    out_ref[0, 1] = x[:, HALF:]


def _feat_mlps(emb, ws, bs):
    n = emb.shape[0]
    return pl.pallas_call(
        _feat_body,
        grid=(2, n // RB),
        in_specs=[
            pl.BlockSpec((RB, DIM), lambda m, r: (r, 0)),
            pl.BlockSpec((1, N_LAYERS, DIM, DIM), lambda m, r: (m, 0, 0, 0)),
            pl.BlockSpec((1, N_LAYERS, DIM), lambda m, r: (m, 0, 0)),
        ],
        out_specs=pl.BlockSpec((1, 2, RB, HALF), lambda m, r: (m, 0, r, 0)),
        out_shape=jax.ShapeDtypeStruct((2, 2, n, HALF), jnp.float32),
    )(emb, ws, bs)


def _upd_body(x_ref, p0_ref, p1_ref, n0_ref, n1_ref, w1_ref, b1_ref,
              w23_ref, b23_ref, out_ref):
    h = jnp.dot(x_ref[...], w1_ref[0:DIM], preferred_element_type=jnp.float32)
    h += jnp.dot(p0_ref[0], w1_ref[DIM:DIM + HALF],
                 preferred_element_type=jnp.float32)
    h += jnp.dot(p1_ref[0], w1_ref[DIM + HALF:2 * DIM],
                 preferred_element_type=jnp.float32)
    h += jnp.dot(n0_ref[0], w1_ref[2 * DIM:2 * DIM + HALF],
                 preferred_element_type=jnp.float32)
    h += jnp.dot(n1_ref[0], w1_ref[2 * DIM + HALF:3 * DIM],
                 preferred_element_type=jnp.float32)
    h += b1_ref[0][None, :]
    h = jnp.maximum(h, 0.0)
    h = jnp.dot(h, w23_ref[0], preferred_element_type=jnp.float32)
    h = jnp.maximum(h + b23_ref[0][None, :], 0.0)
    h = jnp.dot(h, w23_ref[1], preferred_element_type=jnp.float32)
    out_ref[...] = h + b23_ref[1][None, :]


def _upd_mlp(emb, p_aggr, n_aggr, w1, b1, w23, b23):
    n = emb.shape[0]
    half_spec0 = pl.BlockSpec((1, RB, HALF), lambda r: (0, r, 0))
    half_spec1 = pl.BlockSpec((1, RB, HALF), lambda r: (1, r, 0))
    return pl.pallas_call(
        _upd_body,
        grid=(n // RB,),
        in_specs=[
            pl.BlockSpec((RB, DIM), lambda r: (r, 0)),
            half_spec0, half_spec1, half_spec0, half_spec1,
            pl.BlockSpec((3 * DIM, DIM), lambda r: (0, 0)),
            pl.BlockSpec((1, DIM), lambda r: (0, 0)),
            pl.BlockSpec((2, DIM, DIM), lambda r: (0, 0, 0)),
            pl.BlockSpec((2, DIM), lambda r: (0, 0)),
        ],
        out_specs=pl.BlockSpec((RB, DIM), lambda r: (r, 0)),
        out_shape=jax.ShapeDtypeStruct((n, DIM), jnp.float32),
    )(emb, p_aggr, p_aggr, n_aggr, n_aggr, w1, b1, w23, b23)


_MESH = plsc.VectorSubcoreMesh(core_axis_name="c", subcore_axis_name="s")


@functools.partial(
    pl.kernel,
    mesh=_MESH,
    out_type=[
        jax.ShapeDtypeStruct((SC_SUBCORES, NCHUNK, CHUNK), jnp.int32)
        for _ in range(4)
    ],
    scratch_types=[
        pltpu.VMEM((NCHUNK, CHUNK), jnp.int32),
        pltpu.VMEM((NCHUNK, CHUNK), jnp.int32),
        pltpu.VMEM((NCHUNK, CHUNK), jnp.int32),
        pltpu.SemaphoreType.DMA,
    ],
)
def _compose_kernel(v_edge_hbm, c_edge_hbm, p_sel_hbm, n_sel_hbm,
                    vsrc_p_hbm, cdst_p_hbm, vsrc_n_hbm, cdst_n_hbm,
                    sel_v, va_v, ca_v, sem):
    cid = lax.axis_index("c")
    sid = lax.axis_index("s")

    def run(sel_hbm, vsrc_hbm, cdst_hbm):
        pltpu.sync_copy(sel_hbm.at[sid], sel_v)

        def chunk(j, carry):
            pltpu.async_copy(v_edge_hbm.at[sel_v.at[j]], va_v.at[j], sem).wait()
            pltpu.async_copy(c_edge_hbm.at[sel_v.at[j]], ca_v.at[j], sem).wait()
            return carry

        lax.fori_loop(0, NCHUNK, chunk, 0)
        pltpu.sync_copy(va_v, vsrc_hbm.at[sid])
        pltpu.sync_copy(ca_v, cdst_hbm.at[sid])

    @pl.when(cid == 0)
    def _():
        run(p_sel_hbm, vsrc_p_hbm, cdst_p_hbm)

    @pl.when(cid == 1)
    def _():
        run(n_sel_hbm, vsrc_n_hbm, cdst_n_hbm)


@functools.partial(
    pl.kernel,
    mesh=_MESH,
    out_type=jax.ShapeDtypeStruct((SC_CORES, NC_NODES, HALF), jnp.float32),
    scratch_types=[
        pltpu.VMEM((NCHUNK, CHUNK), jnp.int32),
        pltpu.VMEM((NCHUNK, CHUNK), jnp.int32),
        pltpu.VMEM((2, CHUNK, HALF), jnp.float32),
        pltpu.VMEM_SHARED((ACC_ROWS, HALF), jnp.float32),
        pltpu.SemaphoreType.DMA,
        pltpu.SemaphoreType.DMA,
        pltpu.SemaphoreType.DMA,
        pltpu.SemaphoreType.DMA,
        pltpu.SemaphoreType.DMA,
        pltpu.SemaphoreType.DMA,
        pltpu.SemaphoreType.DMA,
        pltpu.SemaphoreType.DMA,
    ],
)
def _aggr_kernel(feat_hbm, src_hbm, dst_hbm, zeros_hbm, out_hbm,
                 src_v, dst_v, bufs_v, acc_sh,
                 isem, zsem, gsem0a, gsem0b, gsem1a, gsem1b, ssem0, ssem1):
    cid = lax.axis_index("c")
    sid = lax.axis_index("s")
    gsem = ((gsem0a, gsem0b), (gsem1a, gsem1b))
    ssem = (ssem0, ssem1)

    def fire_gather(j, b):
        return [
            pltpu.async_copy(
                feat_hbm.at[src_v.at[j, pl.ds(h * 64, 64)]],
                bufs_v.at[b, pl.ds(h * 64, 64)], gsem[b][h])
            for h in range(2)
        ]

    d_src = pltpu.async_copy(src_hbm.at[sid], src_v, isem)
    d_dst = pltpu.async_copy(dst_hbm.at[sid], dst_v, isem)
    zbuf = bufs_v.at[1]
    pltpu.async_copy(zeros_hbm, zbuf, zsem).wait()
    zrows = ACC_ROWS // SC_SUBCORES
    zbase = sid * zrows
    zd = [
        pltpu.async_copy(zbuf, acc_sh.at[pl.ds(zbase + t * CHUNK, CHUNK)],
                         zsem)
        for t in range(zrows // CHUNK)
    ]
    zrem = zrows % CHUNK
    zd.append(pltpu.async_copy(
        zbuf.at[pl.ds(0, zrem)],
        acc_sh.at[pl.ds(zbase + (zrows // CHUNK) * CHUNK, zrem)], zsem))
    d_src.wait()
    d_dst.wait()

    src_off = cid * NC_NODES
    sub_base = sid * (NCHUNK * CHUNK)
    lane = lax.iota(jnp.int32, LANES)

    def fix(i, carry):
        j = i // (CHUNK // LANES)
        k = i % (CHUNK // LANES)
        src_v[j, pl.ds(k * LANES, LANES)] = (
            src_v[j, pl.ds(k * LANES, LANES)] + src_off)
        pos = sub_base + j * CHUNK + k * LANES + lane
        d = dst_v[j, pl.ds(k * LANES, LANES)]
        dst_v[j, pl.ds(k * LANES, LANES)] = jnp.where(
            pos < EP, d, NC_NODES)
        return carry

    lax.fori_loop(0, NCHUNK * (CHUNK // LANES), fix, 0)

    dg = {}
    ds = {}
    dg[0] = fire_gather(0, 0)
    for d in zd:
        d.wait()
    plsc.subcore_barrier()

    for j in range(NCHUNK):
        b = j % 2
        if j + 1 < NCHUNK:
            if j - 1 >= 0:
                ds[j - 1].wait()
            dg[j + 1] = fire_gather(j + 1, 1 - b)
        for d in dg[j]:
            d.wait()
        ds[j] = pltpu.async_copy(
            bufs_v.at[b], acc_sh.at[dst_v.at[j]], ssem[b], add=True)
    ds[NCHUNK - 2].wait()
    ds[NCHUNK - 1].wait()

    plsc.subcore_barrier()

    nfull_out = NC_NODES // CHUNK
    for t in range((nfull_out + SC_SUBCORES - 1) // SC_SUBCORES):
        g = sid + t * SC_SUBCORES
        stage = bufs_v.at[t % 2]

        @pl.when(g < nfull_out)
        def _():
            base = g * CHUNK
            pltpu.sync_copy(acc_sh.at[pl.ds(base, CHUNK)], stage)
            pltpu.sync_copy(stage, out_hbm.at[cid, pl.ds(base, CHUNK)])

    tail = NC_NODES - nfull_out * CHUNK
    @pl.when(sid == nfull_out % SC_SUBCORES)
    def _():
        base = nfull_out * CHUNK
        pltpu.sync_copy(acc_sh.at[pl.ds(base, tail)],
                        bufs_v.at[0, pl.ds(0, tail)])
        pltpu.sync_copy(bufs_v.at[0, pl.ds(0, tail)],
                        out_hbm.at[cid, pl.ds(base, tail)])


def _stack_mlp(params):
    ws = jnp.stack([w for w, _ in params])
    bs = jnp.stack([b for _, b in params])
    return ws, bs


def kernel(v_size, c_size, v_edge_index, c_edge_index, p_edge_index,
           n_edge_index, v_emb, c_emb, params):
    f32 = jnp.float32
    v_emb = v_emb * (jnp.asarray(v_size) == NV).astype(f32)
    c_emb = c_emb * (jnp.asarray(c_size) == NC_NODES).astype(f32)

    def pad_sel(sel):
        return jnp.concatenate(
            [sel, jnp.zeros((EPAD - sel.shape[0],), jnp.int32)]
        ).reshape(SC_SUBCORES, NCHUNK, CHUNK)

    p_sel = pad_sel(p_edge_index)
    n_sel = pad_sel(n_edge_index)
    vsrc_p, cdst_p, vsrc_n, cdst_n = _compose_kernel(
        v_edge_index, c_edge_index, p_sel, n_sel)

    zeros_blk = jnp.zeros((CHUNK, HALF), f32)

    wv_p, bv_p = _stack_mlp(params["p_v2c"])
    wv_n, bv_n = _stack_mlp(params["n_v2c"])
    wc_p, bc_p = _stack_mlp(params["p_c2v"])
    wc_n, bc_n = _stack_mlp(params["n_c2v"])
    wv = jnp.stack([wv_p, wv_n])
    bv = jnp.stack([bv_p, bv_n])
    wc = jnp.stack([wc_p, wc_n])
    bc = jnp.stack([bc_p, bc_n])

    def upd_weights(p):
        (w1, b1), (w2, b2), (w3, b3) = p
        return (w1, b1[None, :], jnp.stack([w2, w3]), jnp.stack([b2, b3]))

    cw = upd_weights(params["c_upd"])
    vw = upd_weights(params["v_upd"])

    v_embs = [v_emb]
    c_embs = [c_emb]
    for _ in range(N_ITER):
        feat_v = _feat_mlps(v_emb, wv, bv)
        feat_c = _feat_mlps(c_emb, wc, bc)

        p_v2c = feat_v[0].reshape(SC_CORES * NV, HALF)
        n_v2c = feat_v[1].reshape(SC_CORES * NV, HALF)
        p_c2v = feat_c[0].reshape(SC_CORES * NC_NODES, HALF)
        n_c2v = feat_c[1].reshape(SC_CORES * NC_NODES, HALF)

        p_v2c_aggr = _aggr_kernel(p_v2c, vsrc_p, cdst_p, zeros_blk)
        n_v2c_aggr = _aggr_kernel(n_v2c, vsrc_n, cdst_n, zeros_blk)
        p_c2v_aggr = _aggr_kernel(p_c2v, cdst_p, vsrc_p, zeros_blk)
        n_c2v_aggr = _aggr_kernel(n_c2v, cdst_n, vsrc_n, zeros_blk)

        c_emb = _upd_mlp(c_emb, p_v2c_aggr, n_v2c_aggr, *cw)
        c_embs.append(c_emb)
        v_emb = _upd_mlp(v_emb, p_c2v_aggr, n_c2v_aggr, *vw)
        v_embs.append(v_emb)

    return (jnp.stack(v_embs), jnp.stack(c_embs))
